# baseline (speedup 1.0000x reference)
import functools

import jax
import jax.numpy as jnp
from jax import lax
from jax.experimental import pallas as pl
from jax.experimental.pallas import tpu as pltpu
from jax.experimental.pallas import tpu_sc as plsc

F = 26
B = 4096
D = 64
ROWS_PER_FIELD = 1000
N = B * F
NC = 2
NS = 16
NW = NC * NS
PER_W = N // NW
GSZ = 128
GPW = PER_W // GSZ
NBUF = 4

_mesh = plsc.VectorSubcoreMesh(core_axis_name="c", subcore_axis_name="s")


@functools.partial(
    pl.kernel,
    out_type=jax.ShapeDtypeStruct((N, D), jnp.float32),
    mesh=_mesh,
    compiler_params=pltpu.CompilerParams(use_tc_tiling_on_sc=False),
    scratch_types=(
        [pltpu.VMEM((PER_W,), jnp.int32)]
        + [pltpu.VMEM((GSZ, D), jnp.float32)] * NBUF
        + [pltpu.SemaphoreType.DMA] * (2 * NBUF)
    ),
)
def _emb_lookup(idx_hbm, table_hbm, out_hbm, idx_v, *bufs_sems):
    bufs = bufs_sems[:NBUF]
    gsems = bufs_sems[NBUF:2 * NBUF]
    wsems = bufs_sems[2 * NBUF:]
    wid = lax.axis_index("s") * NC + lax.axis_index("c")
    base = wid * PER_W
    pltpu.sync_copy(idx_hbm.at[pl.ds(base, PER_W)], idx_v)

    def writeback(g, p):
        pltpu.async_copy(bufs[p], out_hbm.at[pl.ds(base + g * GSZ, GSZ)],
                         wsems[p])

    def wait_writeback(g, p):
        pltpu.make_async_copy(bufs[p], out_hbm.at[pl.ds(base + g * GSZ, GSZ)],
                              wsems[p]).wait()

    for g in range(NBUF):
        pltpu.async_copy(table_hbm.at[idx_v.at[pl.ds(g * GSZ, GSZ)]], bufs[g], gsems[g])

    for g in range(GPW):
        p = g % NBUF
        j = g - 1 + NBUF
        if g >= 1 and j < GPW:
            q = (g - 1) % NBUF
            wait_writeback(g - 1, q)
            pltpu.async_copy(table_hbm.at[idx_v.at[pl.ds(j * GSZ, GSZ)]], bufs[q], gsems[q])
        pltpu.make_async_copy(table_hbm.at[idx_v.at[pl.ds(g * GSZ, GSZ)]], bufs[p],
                              gsems[p]).wait()
        writeback(g, p)

    for g in range(GPW - NBUF, GPW):
        wait_writeback(g, g % NBUF)


_FIELD_OFFS = tuple(f * ROWS_PER_FIELD for f in range(F))


def kernel(x, table):
    offs = jnp.asarray(_FIELD_OFFS, dtype=jnp.int32)
    idx = (x.astype(jnp.int32) + offs[None, :]).reshape(N)
    out = _emb_lookup(idx, table)
    zero = x[0, 0].astype(jnp.float32) * 0.0
    return out.reshape(B, F, D) + zero


# speedup vs baseline: 1.6779x; 1.6779x over previous
import functools

import jax
import jax.numpy as jnp
from jax import lax
from jax.experimental import pallas as pl
from jax.experimental.pallas import tpu as pltpu
from jax.experimental.pallas import tpu_sc as plsc

F = 26
B = 4096
D = 64
ROWS_PER_FIELD = 1000
N = B * F
NC = 2
NS = 16
NW = NC * NS
PER_W = N // NW
GSZ = 128
GPW = PER_W // GSZ

_mesh = plsc.VectorSubcoreMesh(core_axis_name="c", subcore_axis_name="s")

NBUF = 6


@functools.partial(
    pl.kernel,
    out_type=jax.ShapeDtypeStruct((NW, GPW, GSZ, D), jnp.float32),
    mesh=_mesh,
    compiler_params=pltpu.CompilerParams(use_tc_tiling_on_sc=False),
    scratch_types=(
        [pltpu.VMEM((GPW, GSZ), jnp.int32)]
        + [pltpu.VMEM((GSZ, D), jnp.float32)] * NBUF
        + [pltpu.SemaphoreType.DMA] * (2 * NBUF)
    ),
)
def _emb_lookup(x_hbm, table_hbm, out_hbm, idx_v, *bufs_sems):
    bufs = bufs_sems[:NBUF]
    gsems = bufs_sems[NBUF:2 * NBUF]
    wsems = bufs_sems[2 * NBUF:]
    wid = lax.axis_index("s") * NC + lax.axis_index("c")
    pltpu.sync_copy(x_hbm.at[wid], idx_v)

    def add_offsets(g):
        def _add(i, c, g=g):
            sl = pl.ds(i * 16, 16)
            pos = lax.iota(jnp.int32, 16) + (g * GSZ + i * 16)
            idx_v[g, sl] = idx_v[g, sl] + (pos % F) * ROWS_PER_FIELD
            return c
        lax.fori_loop(0, GSZ // 16, _add, 0)

    for g in range(NBUF):
        add_offsets(g)
        pltpu.async_copy(table_hbm.at[idx_v.at[g]], bufs[g], gsems[g])

    for g in range(GPW):
        p = g % NBUF
        j = g - 1 + NBUF
        if g >= 1 and j < GPW:
            q = (g - 1) % NBUF
            add_offsets(j)
            pltpu.make_async_copy(bufs[q], out_hbm.at[wid, g - 1],
                                  wsems[q]).wait()
            pltpu.async_copy(table_hbm.at[idx_v.at[j]], bufs[q], gsems[q])
        pltpu.make_async_copy(table_hbm.at[idx_v.at[g]], bufs[p],
                              gsems[p]).wait()
        pltpu.async_copy(bufs[p], out_hbm.at[wid, g], wsems[p])

    for g in range(GPW - NBUF, GPW):
        p = g % NBUF
        pltpu.make_async_copy(bufs[p], out_hbm.at[wid, g], wsems[p]).wait()


def kernel(x, table):
    x3 = x.astype(jnp.int32).reshape(NW, GPW, GSZ)
    out = _emb_lookup(x3, table)
    return out.reshape(B, F, D)


# speedup vs baseline: 1.6822x; 1.0025x over previous
import functools

import jax
import jax.numpy as jnp
from jax import lax
from jax.experimental import pallas as pl
from jax.experimental.pallas import tpu as pltpu
from jax.experimental.pallas import tpu_sc as plsc

F = 26
B = 4096
D = 64
ROWS_PER_FIELD = 1000
N = B * F
NC = 2
NS = 16
NW = NC * NS
PER_W = N // NW
GSZ = 256
GPW = PER_W // GSZ
NBUF = 6

_mesh = plsc.VectorSubcoreMesh(core_axis_name="c", subcore_axis_name="s")


@functools.partial(
    pl.kernel,
    out_type=jax.ShapeDtypeStruct((NW, GPW, GSZ, D), jnp.float32),
    mesh=_mesh,
    compiler_params=pltpu.CompilerParams(use_tc_tiling_on_sc=False),
    scratch_types=(
        [pltpu.VMEM((GPW, GSZ), jnp.int32)]
        + [pltpu.VMEM((GSZ, D), jnp.float32)] * NBUF
        + [pltpu.SemaphoreType.DMA] * (2 * NBUF)
    ),
)
def _emb_lookup(x_hbm, table_hbm, out_hbm, idx_v, *bufs_sems):
    bufs = bufs_sems[:NBUF]
    gsems = bufs_sems[NBUF:2 * NBUF]
    wsems = bufs_sems[2 * NBUF:]
    wid = lax.axis_index("s") * NC + lax.axis_index("c")
    pltpu.sync_copy(x_hbm.at[wid], idx_v)

    def add_offsets(g):
        def _add(i, c, g=g):
            sl = pl.ds(i * 16, 16)
            pos = lax.iota(jnp.int32, 16) + (g * GSZ + i * 16)
            idx_v[g, sl] = idx_v[g, sl] + (pos % F) * ROWS_PER_FIELD
            return c
        lax.fori_loop(0, GSZ // 16, _add, 0)

    for g in range(min(NBUF, GPW)):
        add_offsets(g)
        pltpu.async_copy(table_hbm.at[idx_v.at[g]], bufs[g], gsems[g])

    for g in range(GPW):
        p = g % NBUF
        j = g - 1 + NBUF
        if g >= 1 and j < GPW:
            q = (g - 1) % NBUF
            add_offsets(j)
            pltpu.make_async_copy(bufs[q], out_hbm.at[wid, g - 1],
                                  wsems[q]).wait()
            pltpu.async_copy(table_hbm.at[idx_v.at[j]], bufs[q], gsems[q])
        pltpu.make_async_copy(table_hbm.at[idx_v.at[g]], bufs[p],
                              gsems[p]).wait()
        pltpu.async_copy(bufs[p], out_hbm.at[wid, g], wsems[p])

    for g in range(max(GPW - NBUF, 0), GPW):
        p = g % NBUF
        pltpu.make_async_copy(bufs[p], out_hbm.at[wid, g], wsems[p]).wait()


def kernel(x, table):
    x3 = x.astype(jnp.int32).reshape(NW, GPW, GSZ)
    out = _emb_lookup(x3, table)
    return out.reshape(B, F, D)


# speedup vs baseline: 1.6839x; 1.0010x over previous
import functools

import jax
import jax.numpy as jnp
from jax import lax
from jax.experimental import pallas as pl
from jax.experimental.pallas import tpu as pltpu
from jax.experimental.pallas import tpu_sc as plsc

F = 26
B = 4096
D = 64
ROWS_PER_FIELD = 1000
N = B * F
NC = 2
NS = 16
NW = NC * NS
PER_W = N // NW
GSZ = 416
GPW = PER_W // GSZ
NBUF = 4

_mesh = plsc.VectorSubcoreMesh(core_axis_name="c", subcore_axis_name="s")


@functools.partial(
    pl.kernel,
    out_type=jax.ShapeDtypeStruct((NW, GPW, GSZ, D), jnp.float32),
    mesh=_mesh,
    compiler_params=pltpu.CompilerParams(use_tc_tiling_on_sc=False),
    scratch_types=(
        [pltpu.VMEM((GPW, GSZ), jnp.int32)]
        + [pltpu.VMEM((GSZ, D), jnp.float32)] * NBUF
        + [pltpu.SemaphoreType.DMA] * (2 * NBUF)
    ),
)
def _emb_lookup(x_hbm, table_hbm, out_hbm, idx_v, *bufs_sems):
    bufs = bufs_sems[:NBUF]
    gsems = bufs_sems[NBUF:2 * NBUF]
    wsems = bufs_sems[2 * NBUF:]
    wid = lax.axis_index("s") * NC + lax.axis_index("c")
    pltpu.sync_copy(x_hbm.at[wid], idx_v)

    def add_offsets(g):
        def _add(i, c, g=g):
            sl = pl.ds(i * 16, 16)
            pos = lax.iota(jnp.int32, 16) + (g * GSZ + i * 16)
            idx_v[g, sl] = idx_v[g, sl] + (pos % F) * ROWS_PER_FIELD
            return c
        lax.fori_loop(0, GSZ // 16, _add, 0)

    for g in range(min(NBUF, GPW)):
        add_offsets(g)
        pltpu.async_copy(table_hbm.at[idx_v.at[g]], bufs[g], gsems[g])

    for g in range(GPW):
        p = g % NBUF
        j = g - 1 + NBUF
        if g >= 1 and j < GPW:
            q = (g - 1) % NBUF
            add_offsets(j)
            pltpu.make_async_copy(bufs[q], out_hbm.at[wid, g - 1],
                                  wsems[q]).wait()
            pltpu.async_copy(table_hbm.at[idx_v.at[j]], bufs[q], gsems[q])
        pltpu.make_async_copy(table_hbm.at[idx_v.at[g]], bufs[p],
                              gsems[p]).wait()
        pltpu.async_copy(bufs[p], out_hbm.at[wid, g], wsems[p])

    for g in range(max(GPW - NBUF, 0), GPW):
        p = g % NBUF
        pltpu.make_async_copy(bufs[p], out_hbm.at[wid, g], wsems[p]).wait()


def kernel(x, table):
    x3 = x.astype(jnp.int32).reshape(NW, GPW, GSZ)
    out = _emb_lookup(x3, table)
    return out.reshape(B, F, D)
